# baseline (speedup 1.0000x reference)
import jax
import jax.numpy as jnp
from jax.experimental import pallas as pl


_BLK_ROWS = 512


def _mask_kernel(x_ref, o_ref):
    x = x_ref[...]
    o_ref[...] = jnp.where(x > 0, x, 0.0)


def kernel(x):
    n_rows, n_cols = x.shape
    grid = (n_rows // _BLK_ROWS,)
    return pl.pallas_call(
        _mask_kernel,
        out_shape=jax.ShapeDtypeStruct(x.shape, x.dtype),
        grid=grid,
        in_specs=[pl.BlockSpec((_BLK_ROWS, n_cols), lambda i: (i, 0))],
        out_specs=pl.BlockSpec((_BLK_ROWS, n_cols), lambda i: (i, 0)),
    )(x)


# speedup vs baseline: 2.7800x; 2.7800x over previous
import jax
import jax.numpy as jnp
from jax.experimental import pallas as pl


_BLK_ROWS = 4096


def _mask_kernel(x_ref, o_ref):
    x = x_ref[...]
    o_ref[...] = jnp.where(x > 0, x, 0.0)


def kernel(x):
    n_rows, n_cols = x.shape
    grid = (n_rows // _BLK_ROWS,)
    return pl.pallas_call(
        _mask_kernel,
        out_shape=jax.ShapeDtypeStruct(x.shape, x.dtype),
        grid=grid,
        in_specs=[pl.BlockSpec((_BLK_ROWS, n_cols), lambda i: (i, 0))],
        out_specs=pl.BlockSpec((_BLK_ROWS, n_cols), lambda i: (i, 0)),
    )(x)


# speedup vs baseline: 3.4479x; 1.2402x over previous
import jax
import jax.numpy as jnp
from jax.experimental import pallas as pl


_BLK_ROWS = 8192


def _mask_kernel(x_ref, o_ref):
    x = x_ref[...]
    o_ref[...] = jnp.where(x > 0, x, 0.0)


def kernel(x):
    n_rows, n_cols = x.shape
    grid = (n_rows // _BLK_ROWS,)
    return pl.pallas_call(
        _mask_kernel,
        out_shape=jax.ShapeDtypeStruct(x.shape, x.dtype),
        grid=grid,
        in_specs=[pl.BlockSpec((_BLK_ROWS, n_cols), lambda i: (i, 0))],
        out_specs=pl.BlockSpec((_BLK_ROWS, n_cols), lambda i: (i, 0)),
    )(x)
